# baseline (speedup 1.0000x reference)
import functools

import jax
import jax.numpy as jnp
from jax.experimental import pallas as pl
from jax.experimental.pallas import tpu as pltpu

INF = 1000000.0
LEVEL_SIZES = [12800, 3200, 800, 208, 56]
RANGES = [(-1.0, 64.0), (64.0, 128.0), (128.0, 256.0), (256.0, 512.0), (512.0, INF)]
P_TOTAL = sum(LEVEL_SIZES)
B = 4
G = 50
C = 80
LN2 = 0.6931471805599453

PBL = 2048
NB = -(-P_TOTAL // PBL)


def _fcos_kernel(loc_ref, cls_ref, box_ref, ctr_ref, gtb_ref, gtl_ref, gta_ref,
                 out_ref, acc_ref):
    k = pl.program_id(0)

    @pl.when(k == 0)
    def _init():
        acc_ref[0] = 0.0
        acc_ref[1] = 0.0
        acc_ref[2] = 0.0
        acc_ref[3] = 0.0

    loc = jnp.transpose(loc_ref[...], (1, 0))
    xs = loc[0:1, :]
    ys = loc[1:2, :]
    pt = k * PBL + jax.lax.broadcasted_iota(jnp.int32, (1, PBL), 1)
    o1, o2, o3, o4 = (LEVEL_SIZES[0],
                      LEVEL_SIZES[0] + LEVEL_SIZES[1],
                      LEVEL_SIZES[0] + LEVEL_SIZES[1] + LEVEL_SIZES[2],
                      LEVEL_SIZES[0] + LEVEL_SIZES[1] + LEVEL_SIZES[2] + LEVEL_SIZES[3])
    lo = jnp.where(pt < o1, RANGES[0][0],
         jnp.where(pt < o2, RANGES[1][0],
         jnp.where(pt < o3, RANGES[2][0],
         jnp.where(pt < o4, RANGES[3][0], RANGES[4][0]))))
    hi = jnp.where(pt < o1, RANGES[0][1],
         jnp.where(pt < o2, RANGES[1][1],
         jnp.where(pt < o3, RANGES[2][1],
         jnp.where(pt < o4, RANGES[3][1], RANGES[4][1]))))
    in_range = pt < P_TOTAL
    col_valid = (k * PBL + jax.lax.broadcasted_iota(jnp.int32, (PBL, 1), 0)
                 ) < P_TOTAL
    n_pad = jnp.maximum((k + 1) * PBL - P_TOTAL, 0).astype(jnp.float32)
    g_iota = jax.lax.broadcasted_iota(jnp.int32, (G, 1), 0)
    lane_c = jax.lax.broadcasted_iota(jnp.int32, (1, C), 1)

    cls_acc = 0.0
    iou_acc = 0.0
    w_acc = 0.0
    ctr_acc = 0.0
    for i in range(B):
        gtb = gtb_ref[i]
        l = xs - gtb[:, 0:1]
        t = ys - gtb[:, 1:2]
        r = gtb[:, 2:3] - xs
        bb = gtb[:, 3:4] - ys
        mn = jnp.minimum(jnp.minimum(l, t), jnp.minimum(r, bb))
        mx = jnp.maximum(jnp.maximum(l, t), jnp.maximum(r, bb))
        valid = (mn > 0.0) & (mn >= lo) & (mx <= hi)
        areas = jnp.where(valid, gta_ref[i], INF)
        min_area = jnp.min(areas, axis=0, keepdims=True)
        min_ind = jnp.min(jnp.where(areas == min_area, g_iota, G),
                          axis=0, keepdims=True)
        pos = (min_area < INF) & in_range
        a_f32 = jnp.where((g_iota == min_ind) & pos, 1.0, 0.0)

        sel = jax.lax.dot_general(gtb, a_f32, (((0,), (0,)), ((), ())),
                                  precision=jax.lax.Precision.HIGHEST,
                                  preferred_element_type=jnp.float32)
        l_t = xs - sel[0:1, :]
        t_t = ys - sel[1:2, :]
        r_t = sel[2:3, :] - xs
        b_t = sel[3:4, :] - ys

        x = jnp.where(col_valid, cls_ref[i], 0.0)
        ax = jnp.abs(x)
        bce_d = (x + ax) * 0.5 + jnp.log1p(jnp.exp(-ax))
        dense_sum = jnp.sum(bce_d) - n_pad * (C * LN2)
        picked = jax.lax.dot_general(a_f32.astype(jnp.bfloat16),
                                     x.astype(jnp.bfloat16),
                                     (((1,), (0,)), ((), ())),
                                     preferred_element_type=jnp.float32)
        m_sel = gtl_ref[i] == lane_c + 1
        pick_sum = jnp.sum(jnp.where(m_sel, picked, 0.0))
        cls_acc += dense_sum - pick_sum

        tl = jnp.where(pos, l_t, 1.0)
        tt = jnp.where(pos, t_t, 1.0)
        tr = jnp.where(pos, r_t, 1.0)
        tb = jnp.where(pos, b_t, 1.0)
        ctr_tgt = ((jnp.minimum(tl, tr) / jnp.maximum(jnp.maximum(tl, tr), 1e-6))
                   * (jnp.minimum(tt, tb) / jnp.maximum(jnp.maximum(tt, tb), 1e-6)))

        bx = jnp.transpose(box_ref[i], (1, 0))
        p_l = jnp.where(pos, jnp.maximum(bx[0:1, :], 0.0), 1.0)
        p_t = jnp.where(pos, jnp.maximum(bx[1:2, :], 0.0), 1.0)
        p_r = jnp.where(pos, jnp.maximum(bx[2:3, :], 0.0), 1.0)
        p_b = jnp.where(pos, jnp.maximum(bx[3:4, :], 0.0), 1.0)
        t_area = (tl + tr) * (tt + tb)
        p_area = (p_l + p_r) * (p_t + p_b)
        a_int = ((jnp.minimum(p_l, tl) + jnp.minimum(p_r, tr))
                 * (jnp.minimum(p_t, tt) + jnp.minimum(p_b, tb)))
        a_union = t_area + p_area - a_int
        ious = (a_int + 1.0) / (a_union + 1.0)
        iou_l = -jnp.log(jnp.maximum(ious, 1e-6))
        iou_acc += jnp.sum(jnp.where(pos, iou_l * ctr_tgt, 0.0))
        w_acc += jnp.sum(jnp.where(pos, ctr_tgt, 0.0))

        cf = ctr_ref[i:i + 1, :]
        ctr_bce = (jnp.maximum(cf, 0.0) - cf * ctr_tgt
                   + jnp.log1p(jnp.exp(-jnp.abs(cf))))
        ctr_acc += jnp.sum(jnp.where(pos, ctr_bce, 0.0))

    acc_ref[0] += cls_acc
    acc_ref[1] += iou_acc
    acc_ref[2] += w_acc
    acc_ref[3] += ctr_acc

    @pl.when(k == NB - 1)
    def _fin():
        lane_o = jax.lax.broadcasted_iota(jnp.int32, (1, 128), 1)
        cls_loss = acc_ref[0] * (1.0 / (B * P_TOTAL * C))
        reg_loss = acc_ref[1] / jnp.maximum(acc_ref[2], 1e-6)
        center_loss = acc_ref[3]
        out_ref[...] = (jnp.where(lane_o == 0, cls_loss, 0.0)
                        + jnp.where(lane_o == 1, reg_loss, 0.0)
                        + jnp.where(lane_o == 2, center_loss, 0.0))


@functools.partial(jax.jit, static_argnames=("interpret",))
def _run(locations, cls, box, centerness, gt_boxes, gt_labels, gt_areas,
         interpret=False):
    gtl = gt_labels.astype(jnp.int32)[:, :, None]
    gta = gt_areas[:, :, None]

    out = pl.pallas_call(
        _fcos_kernel,
        grid=(NB,),
        in_specs=[
            pl.BlockSpec((PBL, 2), lambda k: (k, 0)),
            pl.BlockSpec((B, PBL, C), lambda k: (0, k, 0)),
            pl.BlockSpec((B, PBL, 4), lambda k: (0, k, 0)),
            pl.BlockSpec((B, PBL), lambda k: (0, k)),
            pl.BlockSpec((B, G, 4), lambda k: (0, 0, 0)),
            pl.BlockSpec((B, G, 1), lambda k: (0, 0, 0)),
            pl.BlockSpec((B, G, 1), lambda k: (0, 0, 0)),
        ],
        out_specs=pl.BlockSpec((1, 128), lambda k: (0, 0)),
        out_shape=jax.ShapeDtypeStruct((1, 128), jnp.float32),
        scratch_shapes=[pltpu.SMEM((4,), jnp.float32)],
        compiler_params=pltpu.CompilerParams(
            dimension_semantics=("arbitrary",)),
        interpret=interpret,
    )(locations, cls, box, centerness, gt_boxes, gtl, gta)
    return out[0, 0], out[0, 1], out[0, 2]


def kernel(locations, cls, box, centerness, gt_boxes, gt_labels, gt_areas):
    return _run(locations, cls, box, centerness, gt_boxes, gt_labels, gt_areas)


# speedup vs baseline: 1.4804x; 1.4804x over previous
import functools

import jax
import jax.numpy as jnp
from jax.experimental import pallas as pl
from jax.experimental.pallas import tpu as pltpu

INF = 1000000.0
LEVEL_SIZES = [12800, 3200, 800, 208, 56]
RANGES = [(-1.0, 64.0), (64.0, 128.0), (128.0, 256.0), (256.0, 512.0), (512.0, INF)]
P_TOTAL = sum(LEVEL_SIZES)
B = 4
G = 50
C = 80
LN2 = 0.6931471805599453

PBL = 2048
NB = -(-P_TOTAL // PBL)


def _fcos_kernel(loc_ref, cls_ref, box_ref, ctr_ref, gtb_ref, gtl_ref, gta_ref,
                 out_ref, acc_ref):
    k = pl.program_id(0)

    @pl.when(k == 0)
    def _init():
        acc_ref[0] = 0.0
        acc_ref[1] = 0.0
        acc_ref[2] = 0.0
        acc_ref[3] = 0.0

    xs = loc_ref[0:1, :]
    ys = loc_ref[1:2, :]
    pt = k * PBL + jax.lax.broadcasted_iota(jnp.int32, (1, PBL), 1)
    o1, o2, o3, o4 = (LEVEL_SIZES[0],
                      LEVEL_SIZES[0] + LEVEL_SIZES[1],
                      LEVEL_SIZES[0] + LEVEL_SIZES[1] + LEVEL_SIZES[2],
                      LEVEL_SIZES[0] + LEVEL_SIZES[1] + LEVEL_SIZES[2] + LEVEL_SIZES[3])
    lo = jnp.where(pt < o1, RANGES[0][0],
         jnp.where(pt < o2, RANGES[1][0],
         jnp.where(pt < o3, RANGES[2][0],
         jnp.where(pt < o4, RANGES[3][0], RANGES[4][0]))))
    hi = jnp.where(pt < o1, RANGES[0][1],
         jnp.where(pt < o2, RANGES[1][1],
         jnp.where(pt < o3, RANGES[2][1],
         jnp.where(pt < o4, RANGES[3][1], RANGES[4][1]))))
    in_range = pt < P_TOTAL
    col_valid = (k * PBL + jax.lax.broadcasted_iota(jnp.int32, (PBL, 1), 0)
                 ) < P_TOTAL
    n_pad = jnp.maximum((k + 1) * PBL - P_TOTAL, 0).astype(jnp.float32)
    g_iota = jax.lax.broadcasted_iota(jnp.int32, (G, 1), 0)
    lane_c = jax.lax.broadcasted_iota(jnp.int32, (1, C), 1)

    cls_acc = 0.0
    iou_acc = 0.0
    w_acc = 0.0
    ctr_acc = 0.0
    for i in range(B):
        gtb = gtb_ref[i]
        l = xs - gtb[:, 0:1]
        t = ys - gtb[:, 1:2]
        r = gtb[:, 2:3] - xs
        bb = gtb[:, 3:4] - ys
        mn = jnp.minimum(jnp.minimum(l, t), jnp.minimum(r, bb))
        mx = jnp.maximum(jnp.maximum(l, t), jnp.maximum(r, bb))
        valid = (mn > 0.0) & (mn >= lo) & (mx <= hi)
        areas = jnp.where(valid, gta_ref[i], INF)
        min_area = jnp.min(areas, axis=0, keepdims=True)
        min_ind = jnp.min(jnp.where(areas == min_area, g_iota, G),
                          axis=0, keepdims=True)
        pos = (min_area < INF) & in_range
        a_f32 = jnp.where((g_iota == min_ind) & pos, 1.0, 0.0)

        sel = jax.lax.dot_general(gtb, a_f32, (((0,), (0,)), ((), ())),
                                  preferred_element_type=jnp.float32)
        l_t = xs - sel[0:1, :]
        t_t = ys - sel[1:2, :]
        r_t = sel[2:3, :] - xs
        b_t = sel[3:4, :] - ys

        x = jnp.where(col_valid, cls_ref[i], 0.0)
        ax = jnp.abs(x)
        bce_d = (x + ax) * 0.5 + jnp.log1p(jnp.exp(-ax))
        dense_sum = jnp.sum(bce_d) - n_pad * (C * LN2)
        picked = jax.lax.dot_general(a_f32.astype(jnp.bfloat16),
                                     x.astype(jnp.bfloat16),
                                     (((1,), (0,)), ((), ())),
                                     preferred_element_type=jnp.float32)
        m_sel = gtl_ref[i] == lane_c + 1
        pick_sum = jnp.sum(jnp.where(m_sel, picked, 0.0))
        cls_acc += dense_sum - pick_sum

        tl = jnp.where(pos, l_t, 1.0)
        tt = jnp.where(pos, t_t, 1.0)
        tr = jnp.where(pos, r_t, 1.0)
        tb = jnp.where(pos, b_t, 1.0)
        ctr_tgt = ((jnp.minimum(tl, tr) / jnp.maximum(jnp.maximum(tl, tr), 1e-6))
                   * (jnp.minimum(tt, tb) / jnp.maximum(jnp.maximum(tt, tb), 1e-6)))

        bx = box_ref[i]
        p_l = jnp.where(pos, jnp.maximum(bx[0:1, :], 0.0), 1.0)
        p_t = jnp.where(pos, jnp.maximum(bx[1:2, :], 0.0), 1.0)
        p_r = jnp.where(pos, jnp.maximum(bx[2:3, :], 0.0), 1.0)
        p_b = jnp.where(pos, jnp.maximum(bx[3:4, :], 0.0), 1.0)
        t_area = (tl + tr) * (tt + tb)
        p_area = (p_l + p_r) * (p_t + p_b)
        a_int = ((jnp.minimum(p_l, tl) + jnp.minimum(p_r, tr))
                 * (jnp.minimum(p_t, tt) + jnp.minimum(p_b, tb)))
        a_union = t_area + p_area - a_int
        ious = (a_int + 1.0) / (a_union + 1.0)
        iou_l = -jnp.log(jnp.maximum(ious, 1e-6))
        iou_acc += jnp.sum(jnp.where(pos, iou_l * ctr_tgt, 0.0))
        w_acc += jnp.sum(jnp.where(pos, ctr_tgt, 0.0))

        cf = ctr_ref[i:i + 1, :]
        ctr_bce = (jnp.maximum(cf, 0.0) - cf * ctr_tgt
                   + jnp.log1p(jnp.exp(-jnp.abs(cf))))
        ctr_acc += jnp.sum(jnp.where(pos, ctr_bce, 0.0))

    acc_ref[0] += cls_acc
    acc_ref[1] += iou_acc
    acc_ref[2] += w_acc
    acc_ref[3] += ctr_acc

    @pl.when(k == NB - 1)
    def _fin():
        lane_o = jax.lax.broadcasted_iota(jnp.int32, (1, 128), 1)
        cls_loss = acc_ref[0] * (1.0 / (B * P_TOTAL * C))
        reg_loss = acc_ref[1] / jnp.maximum(acc_ref[2], 1e-6)
        center_loss = acc_ref[3]
        out_ref[...] = (jnp.where(lane_o == 0, cls_loss, 0.0)
                        + jnp.where(lane_o == 1, reg_loss, 0.0)
                        + jnp.where(lane_o == 2, center_loss, 0.0))


@functools.partial(jax.jit, static_argnames=("interpret",))
def _run(locations, cls, box, centerness, gt_boxes, gt_labels, gt_areas,
         interpret=False):
    loc_t = jnp.transpose(locations, (1, 0))
    box_t = jnp.transpose(box, (0, 2, 1))
    gtl = gt_labels.astype(jnp.int32)[:, :, None]
    gta = gt_areas[:, :, None]

    out = pl.pallas_call(
        _fcos_kernel,
        grid=(NB,),
        in_specs=[
            pl.BlockSpec((2, PBL), lambda k: (0, k)),
            pl.BlockSpec((B, PBL, C), lambda k: (0, k, 0)),
            pl.BlockSpec((B, 4, PBL), lambda k: (0, 0, k)),
            pl.BlockSpec((B, PBL), lambda k: (0, k)),
            pl.BlockSpec((B, G, 4), lambda k: (0, 0, 0)),
            pl.BlockSpec((B, G, 1), lambda k: (0, 0, 0)),
            pl.BlockSpec((B, G, 1), lambda k: (0, 0, 0)),
        ],
        out_specs=pl.BlockSpec((1, 128), lambda k: (0, 0)),
        out_shape=jax.ShapeDtypeStruct((1, 128), jnp.float32),
        scratch_shapes=[pltpu.SMEM((4,), jnp.float32)],
        compiler_params=pltpu.CompilerParams(
            dimension_semantics=("arbitrary",)),
        interpret=interpret,
    )(loc_t, cls, box_t, centerness, gt_boxes, gtl, gta)
    return out[0, 0], out[0, 1], out[0, 2]


def kernel(locations, cls, box, centerness, gt_boxes, gt_labels, gt_areas):
    return _run(locations, cls, box, centerness, gt_boxes, gt_labels, gt_areas)
